# Initial kernel scaffold; baseline (speedup 1.0000x reference)
import jax
import jax.numpy as jnp
from jax.experimental import pallas as pl


def kernel(app_embed, api_embed, app_tag_embed, api_tag_embed, app_api_data, app_tag_data, api_tag_data):
    raise NotImplementedError("write your pallas kernel here")


# speedup vs baseline: 2.3315x; 2.3315x over previous
import functools
import jax
import jax.numpy as jnp
from jax import lax
from jax.experimental import pallas as pl
from jax.experimental.pallas import tpu as pltpu
from jax.experimental.pallas import tpu_sc as plsc

_NUM_APP = 20000
_NUM_API = 30000
_NUM_APP_TAG = 500
_NUM_API_TAG = 500
_D = 64
_W = 128
_L = 3

_NC = 2
_NS = 16
_CH = 128


def _sc_scatter_kernel(n_ranges, range_pad, e_pad, deg_mode):
    span = e_pad // _NS
    iters = span // _CH
    rows_per_sub = range_pad // _NS
    per_core = n_ranges // _NC
    mesh = plsc.VectorSubcoreMesh(core_axis_name="c", subcore_axis_name="s")

    def body(table, src_idx, dst_adj, zeros_hbm, ones_hbm, out_hbm,
             idx_s, idx_d, rows_v, acc_sh, sem):
        c = lax.axis_index("c")
        s = lax.axis_index("s")
        r0 = s * rows_per_sub
        if deg_mode:
            pltpu.sync_copy(ones_hbm, rows_v)
        for j in range(per_core):
            r = c * per_core + j
            pltpu.sync_copy(zeros_hbm.at[pl.ds(r0, rows_per_sub)],
                            acc_sh.at[pl.ds(r0, rows_per_sub)])
            plsc.subcore_barrier()

            def step(i, carry):
                base = s * span + i * _CH
                pltpu.sync_copy(dst_adj.at[pl.ds(r * e_pad + base, _CH)],
                                idx_d)
                if not deg_mode:
                    pltpu.sync_copy(src_idx.at[pl.ds(base, _CH)], idx_s)
                    pltpu.async_copy(table.at[idx_s], rows_v, sem).wait()
                pltpu.sync_copy(rows_v, acc_sh.at[idx_d], add=True)
                return carry

            lax.fori_loop(0, iters, step, 0)
            plsc.subcore_barrier()
            pltpu.sync_copy(acc_sh.at[pl.ds(r0, rows_per_sub)],
                            out_hbm.at[r, pl.ds(r0, rows_per_sub)])
            plsc.subcore_barrier()

    return pl.kernel(
        body,
        mesh=mesh,
        out_type=jax.ShapeDtypeStruct((n_ranges, range_pad, _W), jnp.float32),
        scratch_types=[
            pltpu.VMEM((_CH,), jnp.int32),
            pltpu.VMEM((_CH,), jnp.int32),
            pltpu.VMEM((_CH, _W), jnp.float32),
            pltpu.VMEM_SHARED((range_pad, _W), jnp.float32),
            pltpu.SemaphoreType.DMA,
        ],
    )


def _sc_gather3_kernel(e_pad):
    span = e_pad // (_NC * _NS)
    iters = span // _CH
    mesh = plsc.VectorSubcoreMesh(core_axis_name="c", subcore_axis_name="s")

    def body(table, idx3, out_hbm, idx_v, rows_v, sem):
        wid = lax.axis_index("s") * _NC + lax.axis_index("c")

        for t in range(3):
            def step(i, carry):
                base = wid * span + i * _CH
                pltpu.sync_copy(idx3.at[pl.ds(t * e_pad + base, _CH)], idx_v)
                pltpu.async_copy(table.at[idx_v], rows_v, sem).wait()
                pltpu.sync_copy(rows_v, out_hbm.at[t, pl.ds(base, _CH)])
                return carry
            lax.fori_loop(0, iters, step, 0)

    return pl.kernel(
        body,
        mesh=mesh,
        out_type=jax.ShapeDtypeStruct((3, e_pad, _W), jnp.float32),
        scratch_types=[
            pltpu.VMEM((_CH,), jnp.int32),
            pltpu.VMEM((_CH, _W), jnp.float32),
            pltpu.SemaphoreType.DMA,
        ],
    )


def _dis(g):
    return jnp.where(g > 0, lax.rsqrt(jnp.maximum(g, 1.0)), 0.0)


def _scale_kernel(x_ref, g_ref, o_ref):
    o_ref[...] = x_ref[...] * _dis(g_ref[...])


def _scale_acc_kernel(scale, s_ref, g_ref, a_ref, out_ref, acc_ref):
    out = s_ref[...] * _dis(g_ref[...])
    out_ref[...] = out
    acc_ref[...] = (a_ref[...] + out) * scale


def _tc_scale(x, g, blk):
    n, w = x.shape
    return pl.pallas_call(
        _scale_kernel,
        out_shape=jax.ShapeDtypeStruct((n, w), jnp.float32),
        grid=(n // blk,),
        in_specs=[pl.BlockSpec((blk, w), lambda i: (i, 0))] * 2,
        out_specs=pl.BlockSpec((blk, w), lambda i: (i, 0)),
    )(x, g)


def _tc_scale_acc(s, g, a, scale, blk):
    n, w = s.shape
    return pl.pallas_call(
        functools.partial(_scale_acc_kernel, scale),
        out_shape=[jax.ShapeDtypeStruct((n, w), jnp.float32)] * 2,
        grid=(n // blk,),
        in_specs=[pl.BlockSpec((blk, w), lambda i: (i, 0))] * 3,
        out_specs=[pl.BlockSpec((blk, w), lambda i: (i, 0))] * 2,
    )(s, g, a)


def _sumsq_kernel(x_ref, o_ref):
    @pl.when(pl.program_id(0) == 0)
    def _():
        o_ref[...] = jnp.zeros((1, 1), jnp.float32)
    o_ref[...] += jnp.sum(x_ref[...] * x_ref[...]).reshape(1, 1)


def _tc_sumsq(x, blk):
    n, w = x.shape
    return pl.pallas_call(
        _sumsq_kernel,
        out_shape=jax.ShapeDtypeStruct((1, 1), jnp.float32),
        grid=(n // blk,),
        in_specs=[pl.BlockSpec((blk, w), lambda i: (i, 0))],
        out_specs=pl.BlockSpec((1, 1), lambda i: (0, 0)),
    )(x)[0, 0]


def _bpr_kernel(xs_ref, xd_ref, xn_ref, o_ref):
    @pl.when(pl.program_id(0) == 0)
    def _():
        o_ref[...] = jnp.zeros((1, 1), jnp.float32)
    xs = xs_ref[...]
    pos = jnp.sum(xs * xd_ref[...], axis=-1)
    neg = jnp.sum(xs * xn_ref[...], axis=-1)
    z = neg - pos
    sp = jnp.maximum(z, 0.0) + jnp.log(1.0 + jnp.exp(-jnp.abs(z)))
    o_ref[...] += jnp.sum(sp).reshape(1, 1)


def _tc_bpr_sum(xs, xd, xn, blk):
    n, w = xs.shape
    return pl.pallas_call(
        _bpr_kernel,
        out_shape=jax.ShapeDtypeStruct((1, 1), jnp.float32),
        grid=(n // blk,),
        in_specs=[pl.BlockSpec((blk, w), lambda i: (i, 0))] * 3,
        out_specs=pl.BlockSpec((1, 1), lambda i: (0, 0)),
    )(xs, xd, xn)[0, 0]


def _pad_to(v, m):
    return ((v + m - 1) // m) * m


def _lightgcn_sc(x0, src, dst, n, np_, n_ranges, blk):
    rng = n // n_ranges
    range_pad = _pad_to(rng + 8, _NS * 8)
    trash = rng
    e = src.shape[0]
    e_pad = _pad_to(e, _NS * _CH)

    src_p = jnp.zeros((e_pad,), jnp.int32).at[:e].set(src)
    dst_p = jnp.full((e_pad,), jnp.int32(n + 7)).at[:e].set(dst)
    adj = []
    for r in range(n_ranges):
        a = dst_p - r * rng
        a = jnp.where((a >= 0) & (a < rng), a, trash)
        adj.append(a)
    dst_adj = jnp.concatenate(adj).astype(jnp.int32)

    zeros = jnp.zeros((range_pad, _W), jnp.float32)
    ones = jnp.ones((_CH, _W), jnp.float32)

    scat = _sc_scatter_kernel(n_ranges, range_pad, e_pad, deg_mode=False)
    degk = _sc_scatter_kernel(n_ranges, range_pad, e_pad, deg_mode=True)

    def reassemble(o2):
        return jnp.concatenate([o2[r, :rng] for r in range(n_ranges)], axis=0)

    dummy_tab = jnp.zeros((8, _W), jnp.float32)
    degrow = reassemble(degk(dummy_tab, src_p, dst_adj, zeros, ones))
    g = jnp.zeros((np_, _W), jnp.float32).at[:n].set(degrow)

    out = x0
    acc = x0
    for layer in range(_L):
        t = _tc_scale(out, g, blk)
        s2 = scat(t, src_p, dst_adj, zeros, ones)
        s = jnp.zeros((np_, _W), jnp.float32).at[:n].set(reassemble(s2))
        scale = 1.0 / (_L + 1) if layer == _L - 1 else 1.0
        out, acc = _tc_scale_acc(s, g, acc, scale, blk)
    return acc


def kernel(app_embed, api_embed, app_tag_embed, api_tag_embed,
           app_api_data, app_tag_data, api_tag_data):
    n_aa = _NUM_APP + _NUM_API
    np_aa = 50176
    x0 = jnp.zeros((np_aa, _W), jnp.float32)
    x0 = x0.at[:_NUM_APP, :_D].set(app_embed)
    x0 = x0.at[_NUM_APP:n_aa, :_D].set(api_embed)
    x_aa = _lightgcn_sc(x0, app_api_data[0], app_api_data[1],
                        n_aa, np_aa, 4, 1024)

    n_at = _NUM_APP + _NUM_APP_TAG
    np_at = 21504
    x0 = jnp.zeros((np_at, _W), jnp.float32)
    x0 = x0.at[:_NUM_APP, :_D].set(app_embed)
    x0 = x0.at[_NUM_APP:n_at, :_D].set(app_tag_embed)
    x_at = _lightgcn_sc(x0, app_tag_data[0], app_tag_data[1],
                        n_at, np_at, 2, 1024)

    n_it = _NUM_API + _NUM_API_TAG
    np_it = 30720
    x0 = jnp.zeros((np_it, _W), jnp.float32)
    x0 = x0.at[:_NUM_API, :_D].set(api_embed)
    x0 = x0.at[_NUM_API:n_it, :_D].set(api_tag_embed)
    x_it = _lightgcn_sc(x0, api_tag_data[0], api_tag_data[1],
                        n_it, np_it, 4, 1024)

    src = app_api_data[0]
    dst = app_api_data[1]
    e = src.shape[0]
    neg = jax.random.randint(jax.random.key(1), src.shape,
                             _NUM_APP, _NUM_APP + _NUM_API, dtype=jnp.int32)
    e_pad = _pad_to(e, _NC * _NS * _CH)
    idx3 = jnp.zeros((3, e_pad), jnp.int32)
    idx3 = idx3.at[0, :e].set(src).at[1, :e].set(dst).at[2, :e].set(neg)
    idx3 = idx3.reshape(-1)
    rows = _sc_gather3_kernel(e_pad)(x_aa, idx3)

    bpr_sum = _tc_bpr_sum(rows[0, :e], rows[1, :e], rows[2, :e], 8000)
    reg_sum = _tc_sumsq(app_embed, 800) + _tc_sumsq(api_embed, 1000)
    aux = (_tc_sumsq(x_at, 1024) / (n_at * _D)
           + _tc_sumsq(x_it, 1024) / (n_it * _D))
    return bpr_sum / e + 1e-4 * reg_sum / e + 1e-6 * aux
